# Initial kernel scaffold; baseline (speedup 1.0000x reference)
import jax
import jax.numpy as jnp
from jax.experimental import pallas as pl


def kernel(susceptibility, is_infected, infection_time, max_infectiousness, gumbel_u, log_beta, agent_idx, group_idx, now):
    raise NotImplementedError("write your pallas kernel here")


# speedup vs baseline: 181.2551x; 181.2551x over previous
import functools

import jax
import jax.numpy as jnp
from jax import lax
from jax.experimental import pallas as pl
from jax.experimental.pallas import tpu as pltpu
from jax.experimental.pallas import tpu_sc as plsc

N = 100000
E = 1600000
G = 10000
TAU = 0.1
EPS = 1e-10

NC = 2
NS = 16
L = 16
NW = NC * NS
EPT = E // NW
CHUNK = 2000
NCHUNK = EPT // CHUNK
VPC = CHUNK // L

NP = 102400
GP = 10112
RN = NP // 128
RG = GP // 128
RBLK = 80
F32 = jnp.float32


def _stage_a_body(now_ref, logbeta_ref, isinf_ref, maxinf_ref, itime_ref,
                  susc_ref, trans_ref, sb_ref):
    tfi = now_ref[0] - itime_ref[...]
    tpos = jnp.maximum(tfi, 0.0)
    curve = (tpos * tpos) * jnp.exp(-tpos / 2.0)
    trans_ref[...] = (isinf_ref[...] * maxinf_ref[...]
                      * jnp.where(tfi > 0.0, curve, 0.0))
    beta = jnp.exp(logbeta_ref[0])
    sb_ref[...] = (beta * 1.0) * susc_ref[...]


_stage_a = pl.pallas_call(
    _stage_a_body,
    out_shape=[jax.ShapeDtypeStruct((RN, 128), F32)] * 2,
    in_specs=[
        pl.BlockSpec(memory_space=pltpu.SMEM),
        pl.BlockSpec(memory_space=pltpu.SMEM),
        pl.BlockSpec(memory_space=pltpu.VMEM),
        pl.BlockSpec(memory_space=pltpu.VMEM),
        pl.BlockSpec(memory_space=pltpu.VMEM),
        pl.BlockSpec(memory_space=pltpu.VMEM),
    ],
)


def _sc_edge_pass_body(src_hbm, aidx_hbm, gidx_hbm, out_hbm,
                       src_v, acc_v, a_v, g_v, sem, *, gather_agent):
    wid = lax.axis_index("s") * NC + lax.axis_index("c")
    cp = pltpu.async_copy(src_hbm, src_v, sem)
    zeros = jnp.zeros((L,), F32)
    acc_len = acc_v.shape[0]

    def zero_body(i, carry):
        acc_v[pl.ds(i * L, L)] = zeros
        return carry

    lax.fori_loop(0, acc_len // L, zero_body, 0, unroll=8)
    cp.wait()
    base = wid * EPT

    def chunk_body(c, carry):
        pltpu.sync_copy(aidx_hbm.at[pl.ds(base + c * CHUNK, CHUNK)], a_v)
        pltpu.sync_copy(gidx_hbm.at[pl.ds(base + c * CHUNK, CHUNK)], g_v)

        def vec_body(i, inner):
            ai = a_v[pl.ds(i * L, L)]
            gi = g_v[pl.ds(i * L, L)]
            if gather_agent:
                vals = plsc.load_gather(src_v, [ai])
                plsc.addupdate_scatter(acc_v, [gi], vals)
            else:
                vals = plsc.load_gather(src_v, [gi])
                plsc.addupdate_scatter(acc_v, [ai], vals)
            return inner

        lax.fori_loop(0, VPC, vec_body, 0, unroll=4)
        return carry

    lax.fori_loop(0, NCHUNK, chunk_body, 0)
    pltpu.sync_copy(acc_v, out_hbm.at[wid])


@functools.cache
def _sc_kernels():
    mesh = plsc.VectorSubcoreMesh(
        core_axis_name="c", subcore_axis_name="s",
        num_cores=NC, num_subcores=NS)

    def build(src_len, acc_len, gather_agent):
        return pl.kernel(
            functools.partial(_sc_edge_pass_body, gather_agent=gather_agent),
            out_type=jax.ShapeDtypeStruct((NW, acc_len), F32),
            mesh=mesh,
            scratch_types=[
                pltpu.VMEM((src_len,), F32),
                pltpu.VMEM((acc_len,), F32),
                pltpu.VMEM((CHUNK,), jnp.int32),
                pltpu.VMEM((CHUNK,), jnp.int32),
                pltpu.SemaphoreType.DMA,
            ],
            compiler_params=pltpu.CompilerParams(needs_layout_passes=False),
        )

    return build(NP, GP, True), build(GP, NP, False)


def _stage_b_body(part_ref, out_ref):
    out_ref[...] = jnp.sum(part_ref[...], axis=0)


_stage_b = pl.pallas_call(
    _stage_b_body,
    out_shape=jax.ShapeDtypeStruct((RG, 128), F32),
)


def _stage_c_body(now_ref, part_ref, sb_ref, susc_ref, isinf_ref, itime_ref,
                  gu_ref, out_ref):
    s_agent = jnp.sum(part_ref[...], axis=0)
    cum = sb_ref[...] * s_agent
    p = jnp.exp(-cum)
    l0 = jnp.log((1.0 - p) + EPS)
    l1 = jnp.log(p + EPS)
    g0 = -jnp.log(-jnp.log(gu_ref[0] + EPS) + EPS)
    g1 = -jnp.log(-jnp.log(gu_ref[1] + EPS) + EPS)
    x0 = (l0 + g0) / TAU
    x1 = (l1 + g1) / TAU
    m = jnp.maximum(x0, x1)
    e0 = jnp.exp(x0 - m)
    e1 = jnp.exp(x1 - m)
    denom = e0 + e1
    y0 = e0 / denom
    y1 = e1 / denom
    hard0 = jnp.where(y0 >= y1, 1.0, 0.0).astype(F32)
    new_inf = (hard0 - y0) + y0
    out_ref[0] = new_inf
    out_ref[1] = susc_ref[...] - new_inf
    out_ref[2] = isinf_ref[...] + new_inf
    out_ref[3] = jnp.where(new_inf > 0.5, now_ref[0], itime_ref[...])


_stage_c = pl.pallas_call(
    _stage_c_body,
    grid=(RN // RBLK,),
    out_shape=jax.ShapeDtypeStruct((4, RN, 128), F32),
    in_specs=[
        pl.BlockSpec(memory_space=pltpu.SMEM),
        pl.BlockSpec((NW, RBLK, 128), lambda i: (0, i, 0)),
        pl.BlockSpec((RBLK, 128), lambda i: (i, 0)),
        pl.BlockSpec((RBLK, 128), lambda i: (i, 0)),
        pl.BlockSpec((RBLK, 128), lambda i: (i, 0)),
        pl.BlockSpec((RBLK, 128), lambda i: (i, 0)),
        pl.BlockSpec((2, RBLK, 128), lambda i: (0, i, 0)),
    ],
    out_specs=pl.BlockSpec((4, RBLK, 128), lambda i: (0, i, 0)),
)


def _pad2d(x):
    return jnp.pad(x, (0, NP - N)).reshape(RN, 128)


def kernel(susceptibility, is_infected, infection_time, max_infectiousness,
           gumbel_u, log_beta, agent_idx, group_idx, now):
    now_arr = jnp.asarray(now, F32).reshape(1)
    susc2 = _pad2d(susceptibility)
    isinf2 = _pad2d(is_infected)
    itime2 = _pad2d(infection_time)
    maxinf2 = _pad2d(max_infectiousness)
    gu2 = jnp.pad(gumbel_u, ((0, 0), (0, NP - N))).reshape(2, RN, 128)

    sc_groups, sc_agents = _sc_kernels()
    trans2, sb2 = _stage_a(now_arr, log_beta, isinf2, maxinf2, itime2, susc2)
    part_g = sc_groups(trans2.reshape(NP), agent_idx, group_idx)
    gt2 = _stage_b(part_g.reshape(NW, RG, 128))
    part_s = sc_agents(gt2.reshape(GP), agent_idx, group_idx)
    out = _stage_c(now_arr, part_s.reshape(NW, RN, 128), sb2, susc2, isinf2,
                   itime2, gu2)
    return out.reshape(4, NP)[:, :N]


# speedup vs baseline: 261.8185x; 1.4445x over previous
import functools

import jax
import jax.numpy as jnp
from jax import lax
from jax.experimental import pallas as pl
from jax.experimental.pallas import tpu as pltpu
from jax.experimental.pallas import tpu_sc as plsc

N = 100000
E = 1600000
G = 10000
TAU = 0.1
EPS = 1e-10

NC = 2
NS = 16
L = 16
NW = NC * NS
EPT = E // NW
CHUNK = 2000
NCHUNK = EPT // CHUNK
VPC = CHUNK // L

NP = 102400
GP = 10112
RN = NP // 128
RG = GP // 128
RBLK = 80
F32 = jnp.float32


def _stage_a_body(now_ref, logbeta_ref, isinf_ref, maxinf_ref, itime_ref,
                  susc_ref, trans_ref, sb_ref):
    tfi = now_ref[0] - itime_ref[...]
    tpos = jnp.maximum(tfi, 0.0)
    curve = (tpos * tpos) * jnp.exp(-tpos / 2.0)
    trans_ref[...] = (isinf_ref[...] * maxinf_ref[...]
                      * jnp.where(tfi > 0.0, curve, 0.0))
    beta = jnp.exp(logbeta_ref[0])
    sb_ref[...] = (beta * 1.0) * susc_ref[...]


_stage_a = pl.pallas_call(
    _stage_a_body,
    out_shape=[jax.ShapeDtypeStruct((RN, 128), F32)] * 2,
    in_specs=[
        pl.BlockSpec(memory_space=pltpu.SMEM),
        pl.BlockSpec(memory_space=pltpu.SMEM),
        pl.BlockSpec(memory_space=pltpu.VMEM),
        pl.BlockSpec(memory_space=pltpu.VMEM),
        pl.BlockSpec(memory_space=pltpu.VMEM),
        pl.BlockSpec(memory_space=pltpu.VMEM),
    ],
)


def _sc_edge_pass_body(src_hbm, aidx_hbm, gidx_hbm, out_hbm,
                       src_v, acc_v, a0, a1, g0, g1,
                       sem_src, s_a0, s_a1, s_g0, s_g1, *, gather_agent):
    wid = lax.axis_index("s") * NC + lax.axis_index("c")
    cp_src = pltpu.async_copy(src_hbm, src_v, sem_src)
    base = wid * EPT
    abufs, gbufs = [a0, a1], [g0, g1]
    asems, gsems = [s_a0, s_a1], [s_g0, s_g1]

    def issue(c, b):
        sl = pl.ds(base + c * CHUNK, CHUNK)
        return (pltpu.async_copy(aidx_hbm.at[sl], abufs[b], asems[b]),
                pltpu.async_copy(gidx_hbm.at[sl], gbufs[b], gsems[b]))

    pend = [issue(0, 0), issue(1, 1)]
    zeros = jnp.zeros((L,), F32)

    def zero_body(i, carry):
        acc_v[pl.ds(i * L, L)] = zeros
        return carry

    lax.fori_loop(0, acc_v.shape[0] // L, zero_body, 0, unroll=8)
    cp_src.wait()

    for c in range(NCHUNK):
        b = c & 1
        ca, cg = pend[b]
        ca.wait()
        cg.wait()
        a_v, g_v = abufs[b], gbufs[b]

        def vec_body(i, inner, a_v=a_v, g_v=g_v):
            ai = a_v[pl.ds(i * L, L)]
            gi = g_v[pl.ds(i * L, L)]
            if gather_agent:
                vals = plsc.load_gather(src_v, [ai])
                plsc.addupdate_scatter(acc_v, [gi], vals)
            else:
                vals = plsc.load_gather(src_v, [gi])
                plsc.addupdate_scatter(acc_v, [ai], vals)
            return inner

        lax.fori_loop(0, VPC, vec_body, 0, unroll=5)
        if c + 2 < NCHUNK:
            pend[b] = issue(c + 2, b)

    pltpu.sync_copy(acc_v, out_hbm.at[wid])


@functools.cache
def _sc_kernels():
    mesh = plsc.VectorSubcoreMesh(
        core_axis_name="c", subcore_axis_name="s",
        num_cores=NC, num_subcores=NS)

    def build(src_len, acc_len, gather_agent):
        return pl.kernel(
            functools.partial(_sc_edge_pass_body, gather_agent=gather_agent),
            out_type=jax.ShapeDtypeStruct((NW, acc_len), F32),
            mesh=mesh,
            scratch_types=[
                pltpu.VMEM((src_len,), F32),
                pltpu.VMEM((acc_len,), F32),
                pltpu.VMEM((CHUNK,), jnp.int32),
                pltpu.VMEM((CHUNK,), jnp.int32),
                pltpu.VMEM((CHUNK,), jnp.int32),
                pltpu.VMEM((CHUNK,), jnp.int32),
                pltpu.SemaphoreType.DMA,
                pltpu.SemaphoreType.DMA,
                pltpu.SemaphoreType.DMA,
                pltpu.SemaphoreType.DMA,
                pltpu.SemaphoreType.DMA,
            ],
            compiler_params=pltpu.CompilerParams(needs_layout_passes=False),
        )

    return build(NP, GP, True), build(GP, NP, False)


def _stage_b_body(part_ref, out_ref):
    out_ref[...] = jnp.sum(part_ref[...], axis=0)


_stage_b = pl.pallas_call(
    _stage_b_body,
    out_shape=jax.ShapeDtypeStruct((RG, 128), F32),
)


def _stage_c_body(now_ref, part_ref, sb_ref, susc_ref, isinf_ref, itime_ref,
                  gu_ref, out_ref):
    s_agent = jnp.sum(part_ref[...], axis=0)
    cum = sb_ref[...] * s_agent
    p = jnp.exp(-cum)
    l0 = jnp.log((1.0 - p) + EPS)
    l1 = jnp.log(p + EPS)
    g0 = -jnp.log(-jnp.log(gu_ref[0] + EPS) + EPS)
    g1 = -jnp.log(-jnp.log(gu_ref[1] + EPS) + EPS)
    x0 = (l0 + g0) / TAU
    x1 = (l1 + g1) / TAU
    m = jnp.maximum(x0, x1)
    e0 = jnp.exp(x0 - m)
    e1 = jnp.exp(x1 - m)
    denom = e0 + e1
    y0 = e0 / denom
    y1 = e1 / denom
    hard0 = jnp.where(y0 >= y1, 1.0, 0.0).astype(F32)
    new_inf = (hard0 - y0) + y0
    out_ref[0] = new_inf
    out_ref[1] = susc_ref[...] - new_inf
    out_ref[2] = isinf_ref[...] + new_inf
    out_ref[3] = jnp.where(new_inf > 0.5, now_ref[0], itime_ref[...])


_stage_c = pl.pallas_call(
    _stage_c_body,
    grid=(RN // RBLK,),
    out_shape=jax.ShapeDtypeStruct((4, RN, 128), F32),
    in_specs=[
        pl.BlockSpec(memory_space=pltpu.SMEM),
        pl.BlockSpec((NW, RBLK, 128), lambda i: (0, i, 0)),
        pl.BlockSpec((RBLK, 128), lambda i: (i, 0)),
        pl.BlockSpec((RBLK, 128), lambda i: (i, 0)),
        pl.BlockSpec((RBLK, 128), lambda i: (i, 0)),
        pl.BlockSpec((RBLK, 128), lambda i: (i, 0)),
        pl.BlockSpec((2, RBLK, 128), lambda i: (0, i, 0)),
    ],
    out_specs=pl.BlockSpec((4, RBLK, 128), lambda i: (0, i, 0)),
)


def _pad2d(x):
    return jnp.pad(x, (0, NP - N)).reshape(RN, 128)


def kernel(susceptibility, is_infected, infection_time, max_infectiousness,
           gumbel_u, log_beta, agent_idx, group_idx, now):
    now_arr = jnp.asarray(now, F32).reshape(1)
    susc2 = _pad2d(susceptibility)
    isinf2 = _pad2d(is_infected)
    itime2 = _pad2d(infection_time)
    maxinf2 = _pad2d(max_infectiousness)
    gu2 = jnp.pad(gumbel_u, ((0, 0), (0, NP - N))).reshape(2, RN, 128)

    sc_groups, sc_agents = _sc_kernels()
    trans2, sb2 = _stage_a(now_arr, log_beta, isinf2, maxinf2, itime2, susc2)
    part_g = sc_groups(trans2.reshape(NP), agent_idx, group_idx)
    gt2 = _stage_b(part_g.reshape(NW, RG, 128))
    part_s = sc_agents(gt2.reshape(GP), agent_idx, group_idx)
    out = _stage_c(now_arr, part_s.reshape(NW, RN, 128), sb2, susc2, isinf2,
                   itime2, gu2)
    return out.reshape(4, NP)[:, :N]
